# baseline (speedup 1.0000x reference)
import functools

import jax
import jax.numpy as jnp
from jax import lax
from jax.experimental import pallas as pl
from jax.experimental.pallas import tpu as pltpu
from jax.experimental.pallas import tpu_sc as plsc

_R, _C = 2048, 8192
_N = _R * _C
_K = int(0.05 * _N)
_NBUCKETS = 2048
_SHIFT = 20

_NC, _NS = 2, 16
_NW = _NC * _NS
_SCROW = 32768
_NROWS = _N // _SCROW
_ROWS_PER_W = _NROWS // _NW
_CHUNK = 8192
_CHUNKS_PER_ROW = _SCROW // _CHUNK
_VECS = _CHUNK // 16
_HWORDS = 16 * _NBUCKETS
_NHIST = 3

_BLK = 128
_NB = _R // _BLK


def _stage_body(pre_ref, fine_ref, idx_ref):
    tv = fine_ref[...] - pre_ref[...]
    bits = lax.bitcast_convert_type(tv, jnp.int32)
    idx_ref[...] = lax.shift_right_logical(
        jnp.bitwise_and(bits, jnp.int32(0x7FFFFFFF)), _SHIFT
    )


def _hist_body(idx_hbm, out_hbm, b0, b1, h0, h1, h2, s0, s1):
    cid = lax.axis_index("c")
    sid = lax.axis_index("s")
    wid = sid * _NC + cid
    base_row = wid * _ROWS_PER_W

    lanes = lax.iota(jnp.int32, 16)
    ones = jnp.ones((16,), jnp.int32)
    zeros = jnp.zeros((16,), jnp.int32)

    hists = [h0, h1, h2]

    def zbody(c, _):
        for h in hists:
            for u in range(4):
                h[pl.ds(c * 64 + u * 16, 16)] = zeros
        return 0
    lax.fori_loop(0, _HWORDS // 64, zbody, 0)

    bufs = [b0, b1]
    sems = [s0, s1]

    def start(r, slot):
        row = base_row + (r // _CHUNKS_PER_ROW)
        off = (r % _CHUNKS_PER_ROW) * _CHUNK
        return pltpu.async_copy(
            idx_hbm.at[row, pl.ds(off, _CHUNK)], bufs[slot], sems[slot]
        )

    def process(buf_ref):
        def body(i, _):
            for u in range(8):
                bkt = buf_ref[pl.ds(i * 128 + u * 16, 16)]
                plsc.addupdate_scatter(hists[u % 3], [bkt * 16 + lanes], ones)
            return 0
        lax.fori_loop(0, _VECS // 8, body, 0)

    nchunks = _ROWS_PER_W * _CHUNKS_PER_ROW
    pending = [None, None]
    pending[0] = start(0, 0)
    for r in range(nchunks):
        slot = r % 2
        if r + 1 < nchunks:
            pending[(r + 1) % 2] = start(r + 1, (r + 1) % 2)
        pending[slot].wait()
        process(bufs[slot])

    for j, h in enumerate(hists):
        pltpu.sync_copy(h, out_hbm.at[wid, pl.ds(j * _HWORDS, _HWORDS)])


_hist_sc = functools.partial(
    pl.kernel,
    out_type=jax.ShapeDtypeStruct((_NW, _NHIST * _HWORDS), jnp.int32),
    mesh=plsc.VectorSubcoreMesh(core_axis_name="c", subcore_axis_name="s"),
    compiler_params=pltpu.CompilerParams(needs_layout_passes=False),
    scratch_types=[
        pltpu.VMEM((_CHUNK,), jnp.int32),
        pltpu.VMEM((_CHUNK,), jnp.int32),
        pltpu.VMEM((_HWORDS,), jnp.int32),
        pltpu.VMEM((_HWORDS,), jnp.int32),
        pltpu.VMEM((_HWORDS,), jnp.int32),
        pltpu.SemaphoreType.DMA,
        pltpu.SemaphoreType.DMA,
    ],
)(_hist_body)


def _apply_body(pre_ref, fine_ref, hist_ref, out_ref, thr):
    s = pl.program_id(0)

    @pl.when(s == 0)
    def _threshold():
        hr = hist_ref[...]
        a = jnp.sum(hr, axis=0).astype(jnp.float32)
        wi = lax.broadcasted_iota(jnp.int32, (128, 8), 0)
        gi = lax.broadcasted_iota(jnp.int32, (128, 8), 1)
        p = ((wi >> 4) == gi).astype(jnp.float32)
        hb = lax.dot(a, p, precision=lax.Precision.HIGHEST)
        i8 = lax.broadcasted_iota(jnp.int32, (8, 8), 0)
        j8 = lax.broadcasted_iota(jnp.int32, (8, 8), 1)
        u8 = (i8 >= j8).astype(jnp.float32)
        row_suffix = lax.dot(hb, u8, precision=lax.Precision.HIGHEST)
        totals = row_suffix[:, 0:1]
        iq = lax.broadcasted_iota(jnp.int32, (256, 256), 0)
        jq = lax.broadcasted_iota(jnp.int32, (256, 256), 1)
        g = (jq > iq).astype(jnp.float32)
        rstrict = lax.dot(g, totals, precision=lax.Precision.HIGHEST)
        sfx = rstrict + row_suffix
        bstar = jnp.sum((sfx >= jnp.float32(_K)).astype(jnp.int32)) - 1
        thr[0] = bstar * jnp.int32(1 << _SHIFT)

    @pl.when(s > 0)
    def _apply():
        pre = pre_ref[...]
        tv = fine_ref[...] - pre
        bits = lax.bitcast_convert_type(tv, jnp.int32)
        keep = jnp.bitwise_and(bits, jnp.int32(0x7FFFFFFF)) >= thr[0]
        out_ref[...] = pre + jnp.where(keep, tv, 0.0)


@jax.jit
def kernel(pretensor, finetensor):
    blk = pl.BlockSpec((_BLK, _C), lambda b: (b, 0))
    idx = pl.pallas_call(
        _stage_body,
        grid=(_NB,),
        in_specs=[blk, blk],
        out_specs=blk,
        out_shape=jax.ShapeDtypeStruct((_R, _C), jnp.int32),
    )(pretensor, finetensor)

    hist = _hist_sc(idx.reshape(_NROWS, _SCROW))
    hist3 = hist.reshape(_NW * _NHIST, 256, 128)

    data_spec = pl.BlockSpec(
        (_BLK, _C), lambda s: (jnp.maximum(s - 1, 0), 0)
    )
    return pl.pallas_call(
        _apply_body,
        grid=(_NB + 1,),
        in_specs=[
            data_spec,
            data_spec,
            pl.BlockSpec((_NW * _NHIST, 256, 128), lambda s: (0, 0, 0)),
        ],
        out_specs=data_spec,
        out_shape=jax.ShapeDtypeStruct((_R, _C), jnp.float32),
        scratch_shapes=[pltpu.SMEM((1,), jnp.int32)],
    )(pretensor, finetensor, hist3)


# speedup vs baseline: 1.6057x; 1.6057x over previous
import functools

import jax
import jax.numpy as jnp
from jax import lax
from jax.experimental import pallas as pl
from jax.experimental.pallas import tpu as pltpu
from jax.experimental.pallas import tpu_sc as plsc

_R, _C = 2048, 8192
_N = _R * _C
_K = int(0.05 * _N)
_NBUCKETS = 2048
_SHIFT = 20

_NC, _NS = 2, 16
_NW = _NC * _NS
_SCROW = 32768
_NROWS = _N // _SCROW
_ROWS_PER_W = _NROWS // _NW
_CHUNK = 8192
_CHUNKS_PER_ROW = _SCROW // _CHUNK
_VECS = _CHUNK // 16
_HWORDS = 16 * _NBUCKETS
_NHIST = 3

_BLK = 128
_NB = _R // _BLK


def _stage_body(pre_ref, fine_ref, idx_ref):
    tv = fine_ref[...] - pre_ref[...]
    bits = lax.bitcast_convert_type(tv, jnp.int32)
    idx_ref[...] = lax.shift_right_logical(
        jnp.bitwise_and(bits, jnp.int32(0x7FFFFFFF)), _SHIFT
    )


def _hist_body(idx_hbm, out_hbm, b0, b1, h0, h1, h2, s0, s1):
    cid = lax.axis_index("c")
    sid = lax.axis_index("s")
    wid = sid * _NC + cid
    base_row = wid * _ROWS_PER_W

    lanes = lax.iota(jnp.int32, 16)
    ones = jnp.ones((16,), jnp.int32)
    zeros = jnp.zeros((16,), jnp.int32)

    hists = [h0, h1, h2]

    def zbody(c, _):
        for h in hists:
            for u in range(4):
                h[pl.ds(c * 64 + u * 16, 16)] = zeros
        return 0
    lax.fori_loop(0, _HWORDS // 64, zbody, 0)

    bufs = [b0, b1]
    sems = [s0, s1]
    nchunks = _ROWS_PER_W * _CHUNKS_PER_ROW

    def chunk_slice(c):
        row = base_row + lax.shift_right_logical(c, 2)
        off = jnp.bitwise_and(c, _CHUNKS_PER_ROW - 1) * _CHUNK
        return idx_hbm.at[row, pl.ds(off, _CHUNK)]

    def start(c, slot):
        pltpu.async_copy(chunk_slice(c), bufs[slot], sems[slot])

    def process(buf_ref):
        @plsc.parallel_loop(0, _VECS // 8, step=1)
        def body(i):
            for u in range(8):
                bkt = buf_ref[pl.ds(i * 128 + u * 16, 16)]
                plsc.addupdate_scatter(hists[u % 3], [bkt * 16 + lanes], ones)

    def half(c, slot):
        pltpu.make_async_copy(chunk_slice(c), bufs[slot], sems[slot]).wait()
        process(bufs[slot])

        @pl.when(c + 2 < nchunks)
        def _prefetch():
            start(c + 2, slot)

    start(jnp.int32(0), 0)
    start(jnp.int32(1), 1)

    def obody(r, _):
        half(2 * r, 0)
        half(2 * r + 1, 1)
        return 0
    lax.fori_loop(0, nchunks // 2, obody, 0)

    for j, h in enumerate(hists):
        pltpu.sync_copy(h, out_hbm.at[wid, pl.ds(j * _HWORDS, _HWORDS)])


_hist_sc = functools.partial(
    pl.kernel,
    out_type=jax.ShapeDtypeStruct((_NW, _NHIST * _HWORDS), jnp.int32),
    mesh=plsc.VectorSubcoreMesh(core_axis_name="c", subcore_axis_name="s"),
    compiler_params=pltpu.CompilerParams(needs_layout_passes=False),
    scratch_types=[
        pltpu.VMEM((_CHUNK,), jnp.int32),
        pltpu.VMEM((_CHUNK,), jnp.int32),
        pltpu.VMEM((_HWORDS,), jnp.int32),
        pltpu.VMEM((_HWORDS,), jnp.int32),
        pltpu.VMEM((_HWORDS,), jnp.int32),
        pltpu.SemaphoreType.DMA,
        pltpu.SemaphoreType.DMA,
    ],
)(_hist_body)


def _apply_body(pre_ref, fine_ref, hist_ref, out_ref, thr):
    s = pl.program_id(0)

    @pl.when(s == 0)
    def _threshold():
        hr = hist_ref[...]
        a = jnp.sum(hr, axis=0).astype(jnp.float32)
        wi = lax.broadcasted_iota(jnp.int32, (128, 8), 0)
        gi = lax.broadcasted_iota(jnp.int32, (128, 8), 1)
        p = ((wi >> 4) == gi).astype(jnp.float32)
        hb = lax.dot(a, p, precision=lax.Precision.HIGHEST)
        i8 = lax.broadcasted_iota(jnp.int32, (8, 8), 0)
        j8 = lax.broadcasted_iota(jnp.int32, (8, 8), 1)
        u8 = (i8 >= j8).astype(jnp.float32)
        row_suffix = lax.dot(hb, u8, precision=lax.Precision.HIGHEST)
        totals = row_suffix[:, 0:1]
        iq = lax.broadcasted_iota(jnp.int32, (256, 256), 0)
        jq = lax.broadcasted_iota(jnp.int32, (256, 256), 1)
        g = (jq > iq).astype(jnp.float32)
        rstrict = lax.dot(g, totals, precision=lax.Precision.HIGHEST)
        sfx = rstrict + row_suffix
        bstar = jnp.sum((sfx >= jnp.float32(_K)).astype(jnp.int32)) - 1
        thr[0] = bstar * jnp.int32(1 << _SHIFT)

    @pl.when(s > 0)
    def _apply():
        pre = pre_ref[...]
        tv = fine_ref[...] - pre
        bits = lax.bitcast_convert_type(tv, jnp.int32)
        keep = jnp.bitwise_and(bits, jnp.int32(0x7FFFFFFF)) >= thr[0]
        out_ref[...] = pre + jnp.where(keep, tv, 0.0)


@jax.jit
def kernel(pretensor, finetensor):
    blk = pl.BlockSpec((_BLK, _C), lambda b: (b, 0))
    idx = pl.pallas_call(
        _stage_body,
        grid=(_NB,),
        in_specs=[blk, blk],
        out_specs=blk,
        out_shape=jax.ShapeDtypeStruct((_R, _C), jnp.int32),
    )(pretensor, finetensor)

    hist = _hist_sc(idx.reshape(_NROWS, _SCROW))
    hist3 = hist.reshape(_NW * _NHIST, 256, 128)

    data_spec = pl.BlockSpec(
        (_BLK, _C), lambda s: (jnp.maximum(s - 1, 0), 0)
    )
    return pl.pallas_call(
        _apply_body,
        grid=(_NB + 1,),
        in_specs=[
            data_spec,
            data_spec,
            pl.BlockSpec((_NW * _NHIST, 256, 128), lambda s: (0, 0, 0)),
        ],
        out_specs=data_spec,
        out_shape=jax.ShapeDtypeStruct((_R, _C), jnp.float32),
        scratch_shapes=[pltpu.SMEM((1,), jnp.int32)],
    )(pretensor, finetensor, hist3)


# speedup vs baseline: 1.6397x; 1.0211x over previous
import functools

import jax
import jax.numpy as jnp
from jax import lax
from jax.experimental import pallas as pl
from jax.experimental.pallas import tpu as pltpu
from jax.experimental.pallas import tpu_sc as plsc

_R, _C = 2048, 8192
_N = _R * _C
_K = int(0.05 * _N)
_NBUCKETS = 2048
_SHIFT = 20

_NC, _NS = 2, 16
_NW = _NC * _NS
_SCROW = 32768
_NROWS = _N // _SCROW
_NHALF = _NROWS // 2
_ROWS_PER_W = _NHALF // _NW
_CHUNK = 8192
_CHUNKS_PER_ROW = _SCROW // _CHUNK
_VECS = _CHUNK // 16
_HWORDS = 16 * _NBUCKETS
_NHIST = 3

_BLK = 128
_NB = _R // _BLK


def _stage_body(pre_ref, fine_ref, idx_ref):
    tv = fine_ref[...] - pre_ref[...]
    bits = lax.bitcast_convert_type(tv, jnp.int32)
    idx_ref[...] = lax.shift_right_logical(
        jnp.bitwise_and(bits, jnp.int32(0x7FFFFFFF)), _SHIFT
    )


def _hist_body(idx_hbm, out_hbm, b0, b1, h0, h1, h2, s0, s1):
    cid = lax.axis_index("c")
    sid = lax.axis_index("s")
    wid = sid * _NC + cid
    base_row = wid * _ROWS_PER_W

    lanes = lax.iota(jnp.int32, 16)
    ones = jnp.ones((16,), jnp.int32)
    zeros = jnp.zeros((16,), jnp.int32)

    hists = [h0, h1, h2]

    def zbody(c, _):
        for h in hists:
            for u in range(4):
                h[pl.ds(c * 64 + u * 16, 16)] = zeros
        return 0
    lax.fori_loop(0, _HWORDS // 64, zbody, 0)

    bufs = [b0, b1]
    sems = [s0, s1]
    nchunks = _ROWS_PER_W * _CHUNKS_PER_ROW

    def chunk_slice(c):
        row = base_row + lax.shift_right_logical(c, 2)
        off = jnp.bitwise_and(c, _CHUNKS_PER_ROW - 1) * _CHUNK
        return idx_hbm.at[row, pl.ds(off, _CHUNK)]

    def start(c, slot):
        pltpu.async_copy(chunk_slice(c), bufs[slot], sems[slot])

    def process(buf_ref):
        @plsc.parallel_loop(0, _VECS // 8, step=1)
        def body(i):
            for u in range(8):
                bkt = buf_ref[pl.ds(i * 128 + u * 16, 16)]
                plsc.addupdate_scatter(hists[u % 3], [bkt * 16 + lanes], ones)

    def half(c, slot):
        pltpu.make_async_copy(chunk_slice(c), bufs[slot], sems[slot]).wait()
        process(bufs[slot])

        @pl.when(c + 2 < nchunks)
        def _prefetch():
            start(c + 2, slot)

    start(jnp.int32(0), 0)
    start(jnp.int32(1), 1)

    def obody(r, _):
        half(2 * r, 0)
        half(2 * r + 1, 1)
        return 0
    lax.fori_loop(0, nchunks // 2, obody, 0)

    for j, h in enumerate(hists):
        pltpu.sync_copy(h, out_hbm.at[wid, pl.ds(j * _HWORDS, _HWORDS)])


_hist_sc = functools.partial(
    pl.kernel,
    out_type=jax.ShapeDtypeStruct((_NW, _NHIST * _HWORDS), jnp.int32),
    mesh=plsc.VectorSubcoreMesh(core_axis_name="c", subcore_axis_name="s"),
    compiler_params=pltpu.CompilerParams(needs_layout_passes=False),
    scratch_types=[
        pltpu.VMEM((_CHUNK,), jnp.int32),
        pltpu.VMEM((_CHUNK,), jnp.int32),
        pltpu.VMEM((_HWORDS,), jnp.int32),
        pltpu.VMEM((_HWORDS,), jnp.int32),
        pltpu.VMEM((_HWORDS,), jnp.int32),
        pltpu.SemaphoreType.DMA,
        pltpu.SemaphoreType.DMA,
    ],
)(_hist_body)


def _apply_body(pre_ref, fine_ref, hist_ref, hist2_ref, out_ref, thr):
    s = pl.program_id(0)

    @pl.when(s == 0)
    def _threshold():
        hr = hist_ref[...]
        hr2 = hist2_ref[...]
        a = (jnp.sum(hr, axis=0) + jnp.sum(hr2, axis=0)).astype(
            jnp.float32
        )
        wi = lax.broadcasted_iota(jnp.int32, (128, 8), 0)
        gi = lax.broadcasted_iota(jnp.int32, (128, 8), 1)
        p = ((wi >> 4) == gi).astype(jnp.float32)
        hb = lax.dot(a, p, precision=lax.Precision.HIGHEST)
        i8 = lax.broadcasted_iota(jnp.int32, (8, 8), 0)
        j8 = lax.broadcasted_iota(jnp.int32, (8, 8), 1)
        u8 = (i8 >= j8).astype(jnp.float32)
        row_suffix = lax.dot(hb, u8, precision=lax.Precision.HIGHEST)
        totals = row_suffix[:, 0:1]
        iq = lax.broadcasted_iota(jnp.int32, (256, 256), 0)
        jq = lax.broadcasted_iota(jnp.int32, (256, 256), 1)
        g = (jq > iq).astype(jnp.float32)
        rstrict = lax.dot(g, totals, precision=lax.Precision.HIGHEST)
        sfx = rstrict + row_suffix
        bstar = jnp.sum((sfx >= jnp.float32(_K)).astype(jnp.int32)) - 1
        thr[0] = bstar * jnp.int32(1 << _SHIFT)

    @pl.when(s > 0)
    def _apply():
        pre = pre_ref[...]
        tv = fine_ref[...] - pre
        bits = lax.bitcast_convert_type(tv, jnp.int32)
        keep = jnp.bitwise_and(bits, jnp.int32(0x7FFFFFFF)) >= thr[0]
        out_ref[...] = pre + jnp.where(keep, tv, 0.0)


@jax.jit
def kernel(pretensor, finetensor):
    halves = []
    for h in range(2):
        row0 = h * (_NB // 2)
        spec = pl.BlockSpec((_BLK, _C), lambda b, row0=row0: (b + row0, 0))
        halves.append(
            pl.pallas_call(
                _stage_body,
                grid=(_NB // 2,),
                in_specs=[spec, spec],
                out_specs=pl.BlockSpec((_BLK, _C), lambda b: (b, 0)),
                out_shape=jax.ShapeDtypeStruct((_R // 2, _C), jnp.int32),
            )(pretensor, finetensor)
        )

    hists = [
        _hist_sc(idx.reshape(_NHALF, _SCROW)).reshape(
            _NW * _NHIST, 256, 128
        )
        for idx in halves
    ]

    data_spec = pl.BlockSpec(
        (_BLK, _C), lambda s: (jnp.maximum(s - 1, 0), 0)
    )
    hist_spec = pl.BlockSpec(
        (_NW * _NHIST, 256, 128), lambda s: (0, 0, 0)
    )
    return pl.pallas_call(
        _apply_body,
        grid=(_NB + 1,),
        in_specs=[data_spec, data_spec, hist_spec, hist_spec],
        out_specs=data_spec,
        out_shape=jax.ShapeDtypeStruct((_R, _C), jnp.float32),
        scratch_shapes=[pltpu.SMEM((1,), jnp.int32)],
    )(pretensor, finetensor, hists[0], hists[1])
